# initial kernel scaffold (unmeasured)
import jax
import jax.numpy as jnp
from jax import lax
from jax.experimental import pallas as pl
from jax.experimental.pallas import tpu as pltpu


def kernel(
    x,
):
    def body(*refs):
        pass

    out_shape = jax.ShapeDtypeStruct(..., jnp.float32)
    return pl.pallas_call(body, out_shape=out_shape)(...)


# baseline (device time: 297062 ns/iter reference)
import jax
import jax.numpy as jnp
from jax import lax
from jax.experimental import pallas as pl
from jax.experimental.pallas import tpu as pltpu

N_DEV = 4


def kernel(x):
    m_per, n = x.shape
    chunk = m_per // N_DEV

    def body(x_ref, out_ref, comm_ref, send_sems, recv_sems):
        d = lax.axis_index("i")
        left = (d - 1) % N_DEV
        right = (d + 1) % N_DEV

        barrier_sem = pltpu.get_barrier_semaphore()
        for nbr in (left, right):
            pl.semaphore_signal(
                barrier_sem, inc=1,
                device_id=(nbr,), device_id_type=pl.DeviceIdType.MESH,
            )
        pl.semaphore_wait(barrier_sem, 2)

        out_ref[...] = x_ref[...]

        for t in range(2 * (N_DEV - 1)):
            slot = t % 2
            if t < N_DEV - 1:
                send_c = (d - t) % N_DEV
                recv_c = (d - t - 1) % N_DEV
            else:
                s = t - (N_DEV - 1)
                send_c = (d + 1 - s) % N_DEV
                recv_c = (d - s) % N_DEV
            rdma = pltpu.make_async_remote_copy(
                src_ref=out_ref.at[pl.ds(send_c * chunk, chunk), :],
                dst_ref=comm_ref.at[slot],
                send_sem=send_sems.at[slot],
                recv_sem=recv_sems.at[slot],
                device_id=(right,),
                device_id_type=pl.DeviceIdType.MESH,
            )
            rdma.start()
            rdma.wait()
            if t < N_DEV - 1:
                out_ref[pl.ds(recv_c * chunk, chunk), :] += comm_ref[slot]
            else:
                out_ref[pl.ds(recv_c * chunk, chunk), :] = comm_ref[slot]

    return pl.pallas_call(
        body,
        out_shape=jax.ShapeDtypeStruct((m_per, n), x.dtype),
        in_specs=[pl.BlockSpec(memory_space=pltpu.VMEM)],
        out_specs=pl.BlockSpec(memory_space=pltpu.VMEM),
        scratch_shapes=[
            pltpu.VMEM((2, chunk, n), x.dtype),
            pltpu.SemaphoreType.DMA((2,)),
            pltpu.SemaphoreType.DMA((2,)),
        ],
        compiler_params=pltpu.CompilerParams(collective_id=0),
    )(x)


# device time: 162278 ns/iter; 1.8306x vs baseline; 1.8306x over previous
import jax
import jax.numpy as jnp
from jax import lax
from jax.experimental import pallas as pl
from jax.experimental.pallas import tpu as pltpu

N_DEV = 4


def kernel(x):
    m_per, n = x.shape
    chunk = m_per // N_DEV
    n2 = n // 2

    def body(x_ref, out_ref, comm_a, comm_b, send_a, recv_a, send_b, recv_b):
        d = lax.axis_index("i")
        left = (d - 1) % N_DEV
        right = (d + 1) % N_DEV

        barrier_sem = pltpu.get_barrier_semaphore()
        for nbr in (left, right):
            pl.semaphore_signal(
                barrier_sem, inc=1,
                device_id=(nbr,), device_id_type=pl.DeviceIdType.MESH,
            )
        pl.semaphore_wait(barrier_sem, 2)

        out_ref[...] = x_ref[...]

        for t in range(2 * (N_DEV - 1)):
            slot = t % 2
            if t < N_DEV - 1:
                send_ca = (d - t) % N_DEV
                recv_ca = (d - t - 1) % N_DEV
                send_cb = (d + t) % N_DEV
                recv_cb = (d + t + 1) % N_DEV
            else:
                s = t - (N_DEV - 1)
                send_ca = (d + 1 - s) % N_DEV
                recv_ca = (d - s) % N_DEV
                send_cb = (d - 1 + s) % N_DEV
                recv_cb = (d + s) % N_DEV

            rdma_a = pltpu.make_async_remote_copy(
                src_ref=out_ref.at[pl.ds(send_ca * chunk, chunk), pl.ds(0, n2)],
                dst_ref=comm_a.at[slot],
                send_sem=send_a.at[slot],
                recv_sem=recv_a.at[slot],
                device_id=(right,),
                device_id_type=pl.DeviceIdType.MESH,
            )
            rdma_b = pltpu.make_async_remote_copy(
                src_ref=out_ref.at[pl.ds(send_cb * chunk, chunk), pl.ds(n2, n2)],
                dst_ref=comm_b.at[slot],
                send_sem=send_b.at[slot],
                recv_sem=recv_b.at[slot],
                device_id=(left,),
                device_id_type=pl.DeviceIdType.MESH,
            )
            rdma_a.start()
            rdma_b.start()
            rdma_a.wait()
            rdma_b.wait()

            if t < N_DEV - 1:
                out_ref[pl.ds(recv_ca * chunk, chunk), pl.ds(0, n2)] += comm_a[slot]
                out_ref[pl.ds(recv_cb * chunk, chunk), pl.ds(n2, n2)] += comm_b[slot]
            else:
                out_ref[pl.ds(recv_ca * chunk, chunk), pl.ds(0, n2)] = comm_a[slot]
                out_ref[pl.ds(recv_cb * chunk, chunk), pl.ds(n2, n2)] = comm_b[slot]

    return pl.pallas_call(
        body,
        out_shape=jax.ShapeDtypeStruct((m_per, n), x.dtype),
        in_specs=[pl.BlockSpec(memory_space=pltpu.VMEM)],
        out_specs=pl.BlockSpec(memory_space=pltpu.VMEM),
        scratch_shapes=[
            pltpu.VMEM((2, chunk, n2), x.dtype),
            pltpu.VMEM((2, chunk, n2), x.dtype),
            pltpu.SemaphoreType.DMA((2,)),
            pltpu.SemaphoreType.DMA((2,)),
            pltpu.SemaphoreType.DMA((2,)),
            pltpu.SemaphoreType.DMA((2,)),
        ],
        compiler_params=pltpu.CompilerParams(collective_id=0),
    )(x)


# device time: 150630 ns/iter; 1.9721x vs baseline; 1.0773x over previous
import jax
import jax.numpy as jnp
from jax import lax
from jax.experimental import pallas as pl
from jax.experimental.pallas import tpu as pltpu

N_DEV = 4
N_HOP = 2 * (N_DEV - 1)
N_SUB = 2


def kernel(x):
    m_per, n = x.shape
    chunk = m_per // N_DEV
    sub = chunk // N_SUB
    n2 = n // 2

    def send_chunk_a(d, t):
        return (d - t) % N_DEV if t < N_DEV - 1 else (d + 1 - (t - (N_DEV - 1))) % N_DEV

    def recv_chunk_a(d, t):
        return (d - t - 1) % N_DEV if t < N_DEV - 1 else (d - (t - (N_DEV - 1))) % N_DEV

    def send_chunk_b(d, t):
        return (d + t) % N_DEV if t < N_DEV - 1 else (d - 1 + (t - (N_DEV - 1))) % N_DEV

    def recv_chunk_b(d, t):
        return (d + t + 1) % N_DEV if t < N_DEV - 1 else (d + (t - (N_DEV - 1))) % N_DEV

    def body(x_ref, out_ref, comm_a, comm_b,
             send_sems_a, recv_sems_a, send_sems_b, recv_sems_b):
        d = lax.axis_index("i")
        left = (d - 1) % N_DEV
        right = (d + 1) % N_DEV

        barrier_sem = pltpu.get_barrier_semaphore()
        for nbr in (left, right):
            pl.semaphore_signal(
                barrier_sem, inc=1,
                device_id=(nbr,), device_id_type=pl.DeviceIdType.MESH,
            )
        pl.semaphore_wait(barrier_sem, 2)

        def a_rows(t, s):
            return pl.ds(send_chunk_a(d, t) * chunk + s * sub, sub)

        def b_rows(t, s):
            return pl.ds(send_chunk_b(d, t) * chunk + s * sub, sub)

        def make_send(t, s, ring):
            if ring == "a":
                src = out_ref.at[a_rows(t, s), pl.ds(0, n2)]
                dst = comm_a.at[t, s] if t < N_DEV - 1 else src
                return pltpu.make_async_remote_copy(
                    src_ref=src, dst_ref=dst,
                    send_sem=send_sems_a.at[t, s], recv_sem=recv_sems_a.at[t, s],
                    device_id=(right,), device_id_type=pl.DeviceIdType.MESH,
                )
            src = out_ref.at[b_rows(t, s), pl.ds(n2, n2)]
            dst = comm_b.at[t, s] if t < N_DEV - 1 else src
            return pltpu.make_async_remote_copy(
                src_ref=src, dst_ref=dst,
                send_sem=send_sems_b.at[t, s], recv_sem=recv_sems_b.at[t, s],
                device_id=(left,), device_id_type=pl.DeviceIdType.MESH,
            )

        out_ref[pl.ds(d * chunk, chunk), :] = x_ref[pl.ds(d * chunk, chunk), :]
        pending = []
        for s in range(N_SUB):
            for ring in ("a", "b"):
                r = make_send(0, s, ring)
                r.start()
                pending.append(r)
        for j in range(1, N_DEV):
            c = (d + j) % N_DEV
            out_ref[pl.ds(c * chunk, chunk), :] = x_ref[pl.ds(c * chunk, chunk), :]

        for t in range(N_HOP):
            for s in range(N_SUB):
                for ring in ("a", "b"):
                    if ring == "a":
                        recv_sem, comm = recv_sems_a, comm_a
                        rows = pl.ds(recv_chunk_a(d, t) * chunk + s * sub, sub)
                        cols = pl.ds(0, n2)
                        src_nbr = left
                    else:
                        recv_sem, comm = recv_sems_b, comm_b
                        rows = pl.ds(recv_chunk_b(d, t) * chunk + s * sub, sub)
                        cols = pl.ds(n2, n2)
                        src_nbr = right
                    dst = comm.at[t, s] if t < N_DEV - 1 else out_ref.at[rows, cols]
                    recv = pltpu.make_async_remote_copy(
                        src_ref=dst, dst_ref=dst,
                        send_sem=recv_sem.at[t, s], recv_sem=recv_sem.at[t, s],
                        device_id=(src_nbr,), device_id_type=pl.DeviceIdType.MESH,
                    )
                    recv.wait_recv()
                    if t < N_DEV - 1:
                        out_ref[rows, cols] += comm[t, s]
                    if t < N_HOP - 1:
                        r = make_send(t + 1, s, ring)
                        r.start()
                        pending.append(r)

        for r in pending:
            r.wait_send()

    return pl.pallas_call(
        body,
        out_shape=jax.ShapeDtypeStruct((m_per, n), x.dtype),
        in_specs=[pl.BlockSpec(memory_space=pltpu.VMEM)],
        out_specs=pl.BlockSpec(memory_space=pltpu.VMEM),
        scratch_shapes=[
            pltpu.VMEM((N_DEV - 1, N_SUB, sub, n2), x.dtype),
            pltpu.VMEM((N_DEV - 1, N_SUB, sub, n2), x.dtype),
            pltpu.SemaphoreType.DMA((N_HOP, N_SUB)),
            pltpu.SemaphoreType.DMA((N_HOP, N_SUB)),
            pltpu.SemaphoreType.DMA((N_HOP, N_SUB)),
            pltpu.SemaphoreType.DMA((N_HOP, N_SUB)),
        ],
        compiler_params=pltpu.CompilerParams(collective_id=0),
    )(x)
